# baseline (speedup 1.0000x reference)
import functools

import jax
import jax.numpy as jnp
from jax import lax
from jax.experimental import pallas as pl
from jax.experimental.pallas import tpu as pltpu
from jax.experimental.pallas import tpu_sc as plsc

_CH = 256
_BR = 512
_BN_C = 0.9999950000374997


def _agg128(h, srcp, dstp, zeros, n_pad, cpt):
    npt = n_pad // 16
    mesh = plsc.VectorSubcoreMesh(core_axis_name="c", subcore_axis_name="s")

    @functools.partial(
        pl.kernel,
        out_type=jax.ShapeDtypeStruct((2, n_pad, 128), jnp.float32),
        mesh=mesh,
        scratch_types=[
            pltpu.VMEM((_CH,), jnp.int32),
            pltpu.VMEM((_CH,), jnp.int32),
            pltpu.VMEM((_CH, 128), jnp.float32),
            pltpu.VMEM_SHARED((n_pad, 128), jnp.float32),
            pltpu.SemaphoreType.DMA,
        ],
    )
    def k(h_hbm, src_hbm, dst_hbm, z_hbm, out_hbm,
          sidx, didx, rows, acc, gsem):
        c = lax.axis_index("c")
        s = lax.axis_index("s")
        wid = s * 2 + c
        pltpu.sync_copy(z_hbm.at[pl.ds(s * npt, npt)],
                        acc.at[pl.ds(s * npt, npt)])
        plsc.subcore_barrier()

        def body(i, carry):
            base = pl.multiple_of((wid * cpt + i) * _CH, _CH)
            pltpu.sync_copy(src_hbm.at[pl.ds(base, _CH)], sidx)
            pltpu.sync_copy(dst_hbm.at[pl.ds(base, _CH)], didx)
            pltpu.async_copy(h_hbm.at[sidx], rows, gsem).wait()
            pltpu.sync_copy(rows, acc.at[didx], add=True)
            return carry

        lax.fori_loop(0, cpt, body, 0)
        plsc.subcore_barrier()
        pltpu.sync_copy(acc.at[pl.ds(s * npt, npt)],
                        out_hbm.at[c, pl.ds(s * npt, npt)])

    return k(h, srcp, dstp, zeros)


def _layer1_tc(h, p, w1, b1, w2, b2, g, bb, n_pad):
    def body(h_ref, p_ref, w1_ref, b1_ref, w2_ref, b2_ref, g_ref, bb_ref, o_ref):
        m = h_ref[...] + p_ref[0] + p_ref[1]
        t = jnp.maximum(
            jnp.dot(m, w1_ref[...], preferred_element_type=jnp.float32)
            + b1_ref[...], 0.0)
        u = (jnp.dot(t, w2_ref[...], preferred_element_type=jnp.float32)
             + b2_ref[...])
        o_ref[...] = (jnp.maximum(u, 0.0) * (g_ref[...] * _BN_C) + bb_ref[...])

    nb = n_pad // _BR
    return pl.pallas_call(
        body,
        grid=(nb,),
        in_specs=[
            pl.BlockSpec((_BR, 128), lambda i: (i, 0)),
            pl.BlockSpec((2, _BR, 128), lambda i: (0, i, 0)),
            pl.BlockSpec((128, 128), lambda i: (0, 0)),
            pl.BlockSpec((1, 128), lambda i: (0, 0)),
            pl.BlockSpec((128, 128), lambda i: (0, 0)),
            pl.BlockSpec((1, 128), lambda i: (0, 0)),
            pl.BlockSpec((1, 128), lambda i: (0, 0)),
            pl.BlockSpec((1, 128), lambda i: (0, 0)),
        ],
        out_specs=pl.BlockSpec((_BR, 128), lambda i: (i, 0)),
        out_shape=jax.ShapeDtypeStruct((n_pad, 128), jnp.float32),
    )(h, p, w1, b1, w2, b2, g, bb)


def _layer2_tc(h, p, w1, b1, w2, b2, g, bb, n_pad):
    def body(h_ref, p_ref, w1_ref, b1_ref, w2_ref, b2_ref, g_ref, bb_ref, o_ref):
        m = h_ref[...] + p_ref[0] + p_ref[1]
        t = jnp.maximum(
            jnp.dot(m, w1_ref[...], preferred_element_type=jnp.float32)
            + b1_ref[...], 0.0)
        u = (jnp.dot(t, w2_ref[...], preferred_element_type=jnp.float32)
             + b2_ref[...])
        h2 = jnp.maximum(u, 0.0) * (g_ref[...] * _BN_C) + bb_ref[...]
        o_ref[0] = h2[:, :128]
        o_ref[1] = h2[:, 128:]

    nb = n_pad // _BR
    return pl.pallas_call(
        body,
        grid=(nb,),
        in_specs=[
            pl.BlockSpec((_BR, 128), lambda i: (i, 0)),
            pl.BlockSpec((2, _BR, 128), lambda i: (0, i, 0)),
            pl.BlockSpec((128, 256), lambda i: (0, 0)),
            pl.BlockSpec((1, 256), lambda i: (0, 0)),
            pl.BlockSpec((256, 256), lambda i: (0, 0)),
            pl.BlockSpec((1, 256), lambda i: (0, 0)),
            pl.BlockSpec((1, 256), lambda i: (0, 0)),
            pl.BlockSpec((1, 256), lambda i: (0, 0)),
        ],
        out_specs=pl.BlockSpec((2, _BR, 128), lambda i: (0, i, 0)),
        out_shape=jax.ShapeDtypeStruct((2, n_pad, 128), jnp.float32),
    )(h, p, w1, b1, w2, b2, g, bb)


def _layer3_tc(hh, pa, pb, w1, b1, w2, b2, g, bb, n_pad):
    def body(hh_ref, pa_ref, pb_ref, w1_ref, b1_ref, w2_ref, b2_ref,
             g_ref, bb_ref, o_ref):
        m0 = hh_ref[0] + pa_ref[0] + pa_ref[1]
        m1 = hh_ref[1] + pb_ref[0] + pb_ref[1]
        m = jnp.concatenate([m0, m1], axis=1)
        t = jnp.maximum(
            jnp.dot(m, w1_ref[...], preferred_element_type=jnp.float32)
            + b1_ref[...], 0.0)
        u = (jnp.dot(t, w2_ref[...], preferred_element_type=jnp.float32)
             + b2_ref[...])
        o_ref[...] = jnp.maximum(u, 0.0) * (g_ref[...] * _BN_C) + bb_ref[...]

    nb = n_pad // _BR
    return pl.pallas_call(
        body,
        grid=(nb,),
        in_specs=[
            pl.BlockSpec((2, _BR, 128), lambda i: (0, i, 0)),
            pl.BlockSpec((2, _BR, 128), lambda i: (0, i, 0)),
            pl.BlockSpec((2, _BR, 128), lambda i: (0, i, 0)),
            pl.BlockSpec((256, 256), lambda i: (0, 0)),
            pl.BlockSpec((1, 256), lambda i: (0, 0)),
            pl.BlockSpec((256, 256), lambda i: (0, 0)),
            pl.BlockSpec((1, 256), lambda i: (0, 0)),
            pl.BlockSpec((1, 256), lambda i: (0, 0)),
            pl.BlockSpec((1, 256), lambda i: (0, 0)),
        ],
        out_specs=pl.BlockSpec((_BR, 256), lambda i: (i, 0)),
        out_shape=jax.ShapeDtypeStruct((n_pad, 256), jnp.float32),
    )(hh, pa, pb, w1, b1, w2, b2, g, bb)


def _pool_tc(batch_r, h3, n_pad, n_graphs):
    def body(b_ref, h_ref, s_ref, c_ref):
        i = pl.program_id(0)

        @pl.when(i == 0)
        def _():
            s_ref[...] = jnp.zeros_like(s_ref)
            c_ref[...] = jnp.zeros_like(c_ref)

        bvals = b_ref[0]
        ids = lax.broadcasted_iota(jnp.int32, (n_graphs, _BR), 0)
        oh = (ids == bvals).astype(jnp.float32)
        s_ref[...] += jnp.dot(oh, h_ref[...],
                              preferred_element_type=jnp.float32)
        cnt = jnp.sum(oh, axis=1, keepdims=True)
        c_ref[...] += jnp.broadcast_to(cnt, (n_graphs, 128))

    nb = n_pad // _BR
    return pl.pallas_call(
        body,
        grid=(nb,),
        in_specs=[
            pl.BlockSpec((1, 1, _BR), lambda i: (i, 0, 0)),
            pl.BlockSpec((_BR, 256), lambda i: (i, 0)),
        ],
        out_specs=[
            pl.BlockSpec((n_graphs, 256), lambda i: (0, 0)),
            pl.BlockSpec((n_graphs, 128), lambda i: (0, 0)),
        ],
        out_shape=[
            jax.ShapeDtypeStruct((n_graphs, 256), jnp.float32),
            jax.ShapeDtypeStruct((n_graphs, 128), jnp.float32),
        ],
    )(batch_r, h3)


def _head_tc(sums, cnts, gxp, w1a, w1b, b1, w2, b2, w3, b3, n_graphs):
    def body(s_ref, c_ref, gx_ref, w1a_ref, w1b_ref, b1_ref, w2_ref, b2_ref,
             w3_ref, b3_ref, o_ref):
        cnt = jnp.maximum(c_ref[:, 0:1], 1.0)
        z0 = s_ref[...] / cnt
        z1 = jnp.maximum(
            jnp.dot(z0, w1a_ref[...], preferred_element_type=jnp.float32)
            + jnp.dot(gx_ref[...], w1b_ref[...],
                      preferred_element_type=jnp.float32)
            + b1_ref[...], 0.0)
        z2 = jnp.maximum(
            jnp.dot(z1, w2_ref[...], preferred_element_type=jnp.float32)
            + b2_ref[...], 0.0)
        o_ref[...] = (jnp.dot(z2, w3_ref[...],
                              preferred_element_type=jnp.float32)
                      + b3_ref[...])

    full = lambda shape: pl.BlockSpec(shape, lambda: tuple(0 for _ in shape))
    return pl.pallas_call(
        body,
        in_specs=[
            full((n_graphs, 256)), full((n_graphs, 128)),
            full((n_graphs, 128)),
            full((256, 128)), full((128, 128)), full((1, 128)),
            full((128, 128)), full((1, 128)),
            full((128, 128)), full((1, 128)),
        ],
        out_specs=full((n_graphs, 128)),
        out_shape=jax.ShapeDtypeStruct((n_graphs, 128), jnp.float32),
    )(sums, cnts, gxp, w1a, w1b, b1, w2, b2, w3, b3)


def kernel(x, edge_index, batch, global_x, params):
    n, d = x.shape
    e = edge_index.shape[1]
    g_graphs, gd = global_x.shape

    n_pad = ((n + _BR - 1) // _BR) * _BR
    if n_pad == n:
        n_pad += _BR
    cpt = (e + 32 * _CH - 1) // (32 * _CH)
    e_pad = 32 * cpt * _CH

    f32 = jnp.float32
    xp = jnp.pad(x, ((0, n_pad - n), (0, 0)))
    srcp = jnp.concatenate(
        [edge_index[0], jnp.zeros((e_pad - e,), jnp.int32)])
    dstp = jnp.concatenate(
        [edge_index[1], jnp.full((e_pad - e,), n, jnp.int32)])
    zeros = jnp.zeros((n_pad, 128), f32)
    batch_r = jnp.pad(batch, (0, n_pad - n), constant_values=g_graphs)
    batch_r = batch_r.reshape(n_pad // _BR, 1, _BR)

    w11, b11, w12, b12 = params['mlp1']
    w21, b21, w22, b22 = params['mlp2']
    w31, b31, w32, b32 = params['mlp3']
    row = lambda v: v.reshape(1, -1)
    g1, bb1 = row(params['bn1_g']), row(params['bn1_b'])
    g2, bb2 = row(params['bn2_g']), row(params['bn2_b'])
    g3, bb3 = row(params['bn3_g']), row(params['bn3_b'])

    wf1, bf1 = params['Wf1'], row(params['bf1'])
    wf2, bf2 = params['Wf2'], row(params['bf2'])
    wf3, bf3 = params['Wf3'], row(params['bf3'])
    w1a = wf1[:256]
    w1b = jnp.pad(wf1[256:], ((0, 128 - gd), (0, 0)))
    w2p = jnp.pad(wf2, ((0, 0), (0, 128 - wf2.shape[1])))
    b2p = jnp.pad(bf2, ((0, 0), (0, 128 - bf2.shape[1])))
    w3p = jnp.pad(wf3, ((0, 128 - wf3.shape[0]), (0, 128 - wf3.shape[1])))
    b3p = jnp.pad(bf3, ((0, 0), (0, 128 - bf3.shape[1])))
    gxp = jnp.pad(global_x, ((0, 0), (0, 128 - gd)))

    p1 = _agg128(xp, srcp, dstp, zeros, n_pad, cpt)
    h1 = _layer1_tc(xp, p1, w11, row(b11), w12, row(b12), g1, bb1, n_pad)

    p2 = _agg128(h1, srcp, dstp, zeros, n_pad, cpt)
    h2 = _layer2_tc(h1, p2, w21, row(b21), w22, row(b22), g2, bb2, n_pad)

    pa = _agg128(h2[0], srcp, dstp, zeros, n_pad, cpt)
    pb = _agg128(h2[1], srcp, dstp, zeros, n_pad, cpt)
    h3 = _layer3_tc(h2, pa, pb, w31, row(b31), w32, row(b32), g3, bb3, n_pad)

    sums, cnts = _pool_tc(batch_r, h3, n_pad, g_graphs)
    out = _head_tc(sums, cnts, gxp, w1a, w1b, row(bf1), w2p, b2p, w3p, b3p,
                   g_graphs)
    return out[:, :1]


# speedup vs baseline: 1.3614x; 1.3614x over previous
import functools

import jax
import jax.numpy as jnp
from jax import lax
from jax.experimental import pallas as pl
from jax.experimental.pallas import tpu as pltpu
from jax.experimental.pallas import tpu_sc as plsc

_CH = 128
_BR = 512
_BN_C = 0.9999950000374997


def _agg128(h, srcp, dstp, zeros, n_pad, cpt):
    npt = n_pad // 16
    mesh = plsc.VectorSubcoreMesh(core_axis_name="c", subcore_axis_name="s")

    @functools.partial(
        pl.kernel,
        out_type=jax.ShapeDtypeStruct((2, n_pad, 128), jnp.float32),
        mesh=mesh,
        scratch_types=[
            pltpu.VMEM((_CH,), jnp.int32),
            pltpu.VMEM((_CH,), jnp.int32),
            pltpu.VMEM((_CH, 128), jnp.float32),
            pltpu.VMEM_SHARED((n_pad, 128), jnp.float32),
            pltpu.SemaphoreType.DMA,
        ],
    )
    def k(h_hbm, src_hbm, dst_hbm, z_hbm, out_hbm,
          sidx, didx, rows, acc, gsem):
        c = lax.axis_index("c")
        s = lax.axis_index("s")
        wid = s * 2 + c
        pltpu.sync_copy(z_hbm.at[pl.ds(s * npt, npt)],
                        acc.at[pl.ds(s * npt, npt)])
        plsc.subcore_barrier()

        def body(i, carry):
            base = pl.multiple_of((wid * cpt + i) * _CH, _CH)
            pltpu.sync_copy(src_hbm.at[pl.ds(base, _CH)], sidx)
            pltpu.sync_copy(dst_hbm.at[pl.ds(base, _CH)], didx)
            pltpu.async_copy(h_hbm.at[sidx], rows, gsem).wait()
            pltpu.sync_copy(rows, acc.at[didx], add=True)
            return carry

        lax.fori_loop(0, cpt, body, 0)
        plsc.subcore_barrier()
        pltpu.sync_copy(acc.at[pl.ds(s * npt, npt)],
                        out_hbm.at[c, pl.ds(s * npt, npt)])

    return k(h, srcp, dstp, zeros)


def _layer1_tc(h, p, w1, b1, w2, b2, g, bb, n_pad):
    def body(h_ref, p_ref, w1_ref, b1_ref, w2_ref, b2_ref, g_ref, bb_ref, o_ref):
        m = h_ref[...] + p_ref[0] + p_ref[1]
        t = jnp.maximum(
            jnp.dot(m, w1_ref[...], preferred_element_type=jnp.float32)
            + b1_ref[...], 0.0)
        u = (jnp.dot(t, w2_ref[...], preferred_element_type=jnp.float32)
             + b2_ref[...])
        o_ref[...] = (jnp.maximum(u, 0.0) * (g_ref[...] * _BN_C) + bb_ref[...])

    nb = n_pad // _BR
    return pl.pallas_call(
        body,
        grid=(nb,),
        in_specs=[
            pl.BlockSpec((_BR, 128), lambda i: (i, 0)),
            pl.BlockSpec((2, _BR, 128), lambda i: (0, i, 0)),
            pl.BlockSpec((128, 128), lambda i: (0, 0)),
            pl.BlockSpec((1, 128), lambda i: (0, 0)),
            pl.BlockSpec((128, 128), lambda i: (0, 0)),
            pl.BlockSpec((1, 128), lambda i: (0, 0)),
            pl.BlockSpec((1, 128), lambda i: (0, 0)),
            pl.BlockSpec((1, 128), lambda i: (0, 0)),
        ],
        out_specs=pl.BlockSpec((_BR, 128), lambda i: (i, 0)),
        out_shape=jax.ShapeDtypeStruct((n_pad, 128), jnp.float32),
    )(h, p, w1, b1, w2, b2, g, bb)


def _layer2_tc(h, p, w1, b1, w2, b2, g, bb, n_pad):
    def body(h_ref, p_ref, w1_ref, b1_ref, w2_ref, b2_ref, g_ref, bb_ref, o_ref):
        m = h_ref[...] + p_ref[0] + p_ref[1]
        t = jnp.maximum(
            jnp.dot(m, w1_ref[...], preferred_element_type=jnp.float32)
            + b1_ref[...], 0.0)
        u = (jnp.dot(t, w2_ref[...], preferred_element_type=jnp.float32)
             + b2_ref[...])
        h2 = jnp.maximum(u, 0.0) * (g_ref[...] * _BN_C) + bb_ref[...]
        o_ref[0] = h2[:, :128]
        o_ref[1] = h2[:, 128:]

    nb = n_pad // _BR
    return pl.pallas_call(
        body,
        grid=(nb,),
        in_specs=[
            pl.BlockSpec((_BR, 128), lambda i: (i, 0)),
            pl.BlockSpec((2, _BR, 128), lambda i: (0, i, 0)),
            pl.BlockSpec((128, 256), lambda i: (0, 0)),
            pl.BlockSpec((1, 256), lambda i: (0, 0)),
            pl.BlockSpec((256, 256), lambda i: (0, 0)),
            pl.BlockSpec((1, 256), lambda i: (0, 0)),
            pl.BlockSpec((1, 256), lambda i: (0, 0)),
            pl.BlockSpec((1, 256), lambda i: (0, 0)),
        ],
        out_specs=pl.BlockSpec((2, _BR, 128), lambda i: (0, i, 0)),
        out_shape=jax.ShapeDtypeStruct((2, n_pad, 128), jnp.float32),
    )(h, p, w1, b1, w2, b2, g, bb)


def _layer3_tc(hh, pa, pb, w1, b1, w2, b2, g, bb, n_pad):
    def body(hh_ref, pa_ref, pb_ref, w1_ref, b1_ref, w2_ref, b2_ref,
             g_ref, bb_ref, o_ref):
        m0 = hh_ref[0] + pa_ref[0] + pa_ref[1]
        m1 = hh_ref[1] + pb_ref[0] + pb_ref[1]
        m = jnp.concatenate([m0, m1], axis=1)
        t = jnp.maximum(
            jnp.dot(m, w1_ref[...], preferred_element_type=jnp.float32)
            + b1_ref[...], 0.0)
        u = (jnp.dot(t, w2_ref[...], preferred_element_type=jnp.float32)
             + b2_ref[...])
        o_ref[...] = jnp.maximum(u, 0.0) * (g_ref[...] * _BN_C) + bb_ref[...]

    nb = n_pad // _BR
    return pl.pallas_call(
        body,
        grid=(nb,),
        in_specs=[
            pl.BlockSpec((2, _BR, 128), lambda i: (0, i, 0)),
            pl.BlockSpec((2, _BR, 128), lambda i: (0, i, 0)),
            pl.BlockSpec((2, _BR, 128), lambda i: (0, i, 0)),
            pl.BlockSpec((256, 256), lambda i: (0, 0)),
            pl.BlockSpec((1, 256), lambda i: (0, 0)),
            pl.BlockSpec((256, 256), lambda i: (0, 0)),
            pl.BlockSpec((1, 256), lambda i: (0, 0)),
            pl.BlockSpec((1, 256), lambda i: (0, 0)),
            pl.BlockSpec((1, 256), lambda i: (0, 0)),
        ],
        out_specs=pl.BlockSpec((_BR, 256), lambda i: (i, 0)),
        out_shape=jax.ShapeDtypeStruct((n_pad, 256), jnp.float32),
    )(hh, pa, pb, w1, b1, w2, b2, g, bb)


def _pool_tc(batch_r, h3, n_pad, n_graphs):
    def body(b_ref, h_ref, s_ref, c_ref):
        i = pl.program_id(0)

        @pl.when(i == 0)
        def _():
            s_ref[...] = jnp.zeros_like(s_ref)
            c_ref[...] = jnp.zeros_like(c_ref)

        bvals = b_ref[0]
        ids = lax.broadcasted_iota(jnp.int32, (n_graphs, _BR), 0)
        oh = (ids == bvals).astype(jnp.float32)
        s_ref[...] += jnp.dot(oh, h_ref[...],
                              preferred_element_type=jnp.float32)
        cnt = jnp.sum(oh, axis=1, keepdims=True)
        c_ref[...] += jnp.broadcast_to(cnt, (n_graphs, 128))

    nb = n_pad // _BR
    return pl.pallas_call(
        body,
        grid=(nb,),
        in_specs=[
            pl.BlockSpec((1, 1, _BR), lambda i: (i, 0, 0)),
            pl.BlockSpec((_BR, 256), lambda i: (i, 0)),
        ],
        out_specs=[
            pl.BlockSpec((n_graphs, 256), lambda i: (0, 0)),
            pl.BlockSpec((n_graphs, 128), lambda i: (0, 0)),
        ],
        out_shape=[
            jax.ShapeDtypeStruct((n_graphs, 256), jnp.float32),
            jax.ShapeDtypeStruct((n_graphs, 128), jnp.float32),
        ],
    )(batch_r, h3)


def _head_tc(sums, cnts, gxp, w1a, w1b, b1, w2, b2, w3, b3, n_graphs):
    def body(s_ref, c_ref, gx_ref, w1a_ref, w1b_ref, b1_ref, w2_ref, b2_ref,
             w3_ref, b3_ref, o_ref):
        cnt = jnp.maximum(c_ref[:, 0:1], 1.0)
        z0 = s_ref[...] / cnt
        z1 = jnp.maximum(
            jnp.dot(z0, w1a_ref[...], preferred_element_type=jnp.float32)
            + jnp.dot(gx_ref[...], w1b_ref[...],
                      preferred_element_type=jnp.float32)
            + b1_ref[...], 0.0)
        z2 = jnp.maximum(
            jnp.dot(z1, w2_ref[...], preferred_element_type=jnp.float32)
            + b2_ref[...], 0.0)
        o_ref[...] = (jnp.dot(z2, w3_ref[...],
                              preferred_element_type=jnp.float32)
                      + b3_ref[...])

    full = lambda shape: pl.BlockSpec(shape, lambda: tuple(0 for _ in shape))
    return pl.pallas_call(
        body,
        in_specs=[
            full((n_graphs, 256)), full((n_graphs, 128)),
            full((n_graphs, 128)),
            full((256, 128)), full((128, 128)), full((1, 128)),
            full((128, 128)), full((1, 128)),
            full((128, 128)), full((1, 128)),
        ],
        out_specs=full((n_graphs, 128)),
        out_shape=jax.ShapeDtypeStruct((n_graphs, 128), jnp.float32),
    )(sums, cnts, gxp, w1a, w1b, b1, w2, b2, w3, b3)


def kernel(x, edge_index, batch, global_x, params):
    n, d = x.shape
    e = edge_index.shape[1]
    g_graphs, gd = global_x.shape

    n_pad = ((n + _BR - 1) // _BR) * _BR
    if n_pad == n:
        n_pad += _BR
    cpt = (e + 32 * _CH - 1) // (32 * _CH)
    e_pad = 32 * cpt * _CH

    f32 = jnp.float32
    xp = jnp.pad(x, ((0, n_pad - n), (0, 0)))
    srcp = jnp.concatenate(
        [edge_index[0], jnp.zeros((e_pad - e,), jnp.int32)])
    dstp = jnp.concatenate(
        [edge_index[1], jnp.full((e_pad - e,), n, jnp.int32)])
    zeros = jnp.zeros((n_pad, 128), f32)
    batch_r = jnp.pad(batch, (0, n_pad - n), constant_values=g_graphs)
    batch_r = batch_r.reshape(n_pad // _BR, 1, _BR)

    w11, b11, w12, b12 = params['mlp1']
    w21, b21, w22, b22 = params['mlp2']
    w31, b31, w32, b32 = params['mlp3']
    row = lambda v: v.reshape(1, -1)
    g1, bb1 = row(params['bn1_g']), row(params['bn1_b'])
    g2, bb2 = row(params['bn2_g']), row(params['bn2_b'])
    g3, bb3 = row(params['bn3_g']), row(params['bn3_b'])

    wf1, bf1 = params['Wf1'], row(params['bf1'])
    wf2, bf2 = params['Wf2'], row(params['bf2'])
    wf3, bf3 = params['Wf3'], row(params['bf3'])
    w1a = wf1[:256]
    w1b = jnp.pad(wf1[256:], ((0, 128 - gd), (0, 0)))
    w2p = jnp.pad(wf2, ((0, 0), (0, 128 - wf2.shape[1])))
    b2p = jnp.pad(bf2, ((0, 0), (0, 128 - bf2.shape[1])))
    w3p = jnp.pad(wf3, ((0, 128 - wf3.shape[0]), (0, 128 - wf3.shape[1])))
    b3p = jnp.pad(bf3, ((0, 0), (0, 128 - bf3.shape[1])))
    gxp = jnp.pad(global_x, ((0, 0), (0, 128 - gd)))

    p1 = _agg128(xp, srcp, dstp, zeros, n_pad, cpt)
    h1 = _layer1_tc(xp, p1, w11, row(b11), w12, row(b12), g1, bb1, n_pad)

    p2 = _agg128(h1, srcp, dstp, zeros, n_pad, cpt)
    h2 = _layer2_tc(h1, p2, w21, row(b21), w22, row(b22), g2, bb2, n_pad)

    pa = _agg128(h2[0], srcp, dstp, zeros, n_pad, cpt)
    pb = _agg128(h2[1], srcp, dstp, zeros, n_pad, cpt)
    h3 = _layer3_tc(h2, pa, pb, w31, row(b31), w32, row(b32), g3, bb3, n_pad)

    sums, cnts = _pool_tc(batch_r, h3, n_pad, g_graphs)
    out = _head_tc(sums, cnts, gxp, w1a, w1b, row(bf1), w2p, b2p, w3p, b3p,
                   g_graphs)
    return out[:, :1]


# speedup vs baseline: 1.5056x; 1.1059x over previous
import functools

import jax
import jax.numpy as jnp
from jax import lax
from jax.experimental import pallas as pl
from jax.experimental.pallas import tpu as pltpu
from jax.experimental.pallas import tpu_sc as plsc

_CH = 128
_SPLIT0 = 60
_BR = 512
_BN_C = 0.9999950000374997


def _agg128(h, srcp, dstp, zeros, n_pad, cpt):
    npt = n_pad // 16
    cpt0 = (2 * cpt * _SPLIT0 + 50) // 100
    cpt1 = 2 * cpt - cpt0
    mesh = plsc.VectorSubcoreMesh(core_axis_name="c", subcore_axis_name="s")

    @functools.partial(
        pl.kernel,
        out_type=jax.ShapeDtypeStruct((2, n_pad, 128), jnp.float32),
        mesh=mesh,
        scratch_types=[
            pltpu.VMEM((_CH,), jnp.int32),
            pltpu.VMEM((_CH,), jnp.int32),
            pltpu.VMEM((_CH, 128), jnp.float32),
            pltpu.VMEM_SHARED((n_pad, 128), jnp.float32),
            pltpu.SemaphoreType.DMA,
        ],
    )
    def k(h_hbm, src_hbm, dst_hbm, z_hbm, out_hbm,
          sidx, didx, rows, acc, gsem):
        c = lax.axis_index("c")
        s = lax.axis_index("s")
        pltpu.sync_copy(z_hbm.at[pl.ds(s * npt, npt)],
                        acc.at[pl.ds(s * npt, npt)])
        plsc.subcore_barrier()

        tb = s * (cpt0 + cpt1) + c * cpt0
        tn = jnp.where(c == 0, cpt0, cpt1)

        def body(i, carry):
            base = pl.multiple_of((tb + i) * _CH, _CH)
            pltpu.sync_copy(src_hbm.at[pl.ds(base, _CH)], sidx)
            pltpu.sync_copy(dst_hbm.at[pl.ds(base, _CH)], didx)
            pltpu.async_copy(h_hbm.at[sidx], rows, gsem).wait()
            pltpu.sync_copy(rows, acc.at[didx], add=True)
            return carry

        lax.fori_loop(0, tn, body, 0)
        plsc.subcore_barrier()
        pltpu.sync_copy(acc.at[pl.ds(s * npt, npt)],
                        out_hbm.at[c, pl.ds(s * npt, npt)])

    return k(h, srcp, dstp, zeros)


def _layer1_tc(h, p, w1, b1, w2, b2, g, bb, n_pad):
    def body(h_ref, p_ref, w1_ref, b1_ref, w2_ref, b2_ref, g_ref, bb_ref, o_ref):
        m = h_ref[...] + p_ref[0] + p_ref[1]
        t = jnp.maximum(
            jnp.dot(m, w1_ref[...], preferred_element_type=jnp.float32)
            + b1_ref[...], 0.0)
        u = (jnp.dot(t, w2_ref[...], preferred_element_type=jnp.float32)
             + b2_ref[...])
        o_ref[...] = (jnp.maximum(u, 0.0) * (g_ref[...] * _BN_C) + bb_ref[...])

    nb = n_pad // _BR
    return pl.pallas_call(
        body,
        grid=(nb,),
        in_specs=[
            pl.BlockSpec((_BR, 128), lambda i: (i, 0)),
            pl.BlockSpec((2, _BR, 128), lambda i: (0, i, 0)),
            pl.BlockSpec((128, 128), lambda i: (0, 0)),
            pl.BlockSpec((1, 128), lambda i: (0, 0)),
            pl.BlockSpec((128, 128), lambda i: (0, 0)),
            pl.BlockSpec((1, 128), lambda i: (0, 0)),
            pl.BlockSpec((1, 128), lambda i: (0, 0)),
            pl.BlockSpec((1, 128), lambda i: (0, 0)),
        ],
        out_specs=pl.BlockSpec((_BR, 128), lambda i: (i, 0)),
        out_shape=jax.ShapeDtypeStruct((n_pad, 128), jnp.float32),
    )(h, p, w1, b1, w2, b2, g, bb)


def _layer2_tc(h, p, w1, b1, w2, b2, g, bb, n_pad):
    def body(h_ref, p_ref, w1_ref, b1_ref, w2_ref, b2_ref, g_ref, bb_ref, o_ref):
        m = h_ref[...] + p_ref[0] + p_ref[1]
        t = jnp.maximum(
            jnp.dot(m, w1_ref[...], preferred_element_type=jnp.float32)
            + b1_ref[...], 0.0)
        u = (jnp.dot(t, w2_ref[...], preferred_element_type=jnp.float32)
             + b2_ref[...])
        h2 = jnp.maximum(u, 0.0) * (g_ref[...] * _BN_C) + bb_ref[...]
        o_ref[0] = h2[:, :128]
        o_ref[1] = h2[:, 128:]

    nb = n_pad // _BR
    return pl.pallas_call(
        body,
        grid=(nb,),
        in_specs=[
            pl.BlockSpec((_BR, 128), lambda i: (i, 0)),
            pl.BlockSpec((2, _BR, 128), lambda i: (0, i, 0)),
            pl.BlockSpec((128, 256), lambda i: (0, 0)),
            pl.BlockSpec((1, 256), lambda i: (0, 0)),
            pl.BlockSpec((256, 256), lambda i: (0, 0)),
            pl.BlockSpec((1, 256), lambda i: (0, 0)),
            pl.BlockSpec((1, 256), lambda i: (0, 0)),
            pl.BlockSpec((1, 256), lambda i: (0, 0)),
        ],
        out_specs=pl.BlockSpec((2, _BR, 128), lambda i: (0, i, 0)),
        out_shape=jax.ShapeDtypeStruct((2, n_pad, 128), jnp.float32),
    )(h, p, w1, b1, w2, b2, g, bb)


def _layer3_tc(hh, pa, pb, w1, b1, w2, b2, g, bb, n_pad):
    def body(hh_ref, pa_ref, pb_ref, w1_ref, b1_ref, w2_ref, b2_ref,
             g_ref, bb_ref, o_ref):
        m0 = hh_ref[0] + pa_ref[0] + pa_ref[1]
        m1 = hh_ref[1] + pb_ref[0] + pb_ref[1]
        m = jnp.concatenate([m0, m1], axis=1)
        t = jnp.maximum(
            jnp.dot(m, w1_ref[...], preferred_element_type=jnp.float32)
            + b1_ref[...], 0.0)
        u = (jnp.dot(t, w2_ref[...], preferred_element_type=jnp.float32)
             + b2_ref[...])
        o_ref[...] = jnp.maximum(u, 0.0) * (g_ref[...] * _BN_C) + bb_ref[...]

    nb = n_pad // _BR
    return pl.pallas_call(
        body,
        grid=(nb,),
        in_specs=[
            pl.BlockSpec((2, _BR, 128), lambda i: (0, i, 0)),
            pl.BlockSpec((2, _BR, 128), lambda i: (0, i, 0)),
            pl.BlockSpec((2, _BR, 128), lambda i: (0, i, 0)),
            pl.BlockSpec((256, 256), lambda i: (0, 0)),
            pl.BlockSpec((1, 256), lambda i: (0, 0)),
            pl.BlockSpec((256, 256), lambda i: (0, 0)),
            pl.BlockSpec((1, 256), lambda i: (0, 0)),
            pl.BlockSpec((1, 256), lambda i: (0, 0)),
            pl.BlockSpec((1, 256), lambda i: (0, 0)),
        ],
        out_specs=pl.BlockSpec((_BR, 256), lambda i: (i, 0)),
        out_shape=jax.ShapeDtypeStruct((n_pad, 256), jnp.float32),
    )(hh, pa, pb, w1, b1, w2, b2, g, bb)


def _pool_tc(batch_r, h3, n_pad, n_graphs):
    def body(b_ref, h_ref, s_ref, c_ref):
        i = pl.program_id(0)

        @pl.when(i == 0)
        def _():
            s_ref[...] = jnp.zeros_like(s_ref)
            c_ref[...] = jnp.zeros_like(c_ref)

        bvals = b_ref[0]
        ids = lax.broadcasted_iota(jnp.int32, (n_graphs, _BR), 0)
        oh = (ids == bvals).astype(jnp.float32)
        s_ref[...] += jnp.dot(oh, h_ref[...],
                              preferred_element_type=jnp.float32)
        cnt = jnp.sum(oh, axis=1, keepdims=True)
        c_ref[...] += jnp.broadcast_to(cnt, (n_graphs, 128))

    nb = n_pad // _BR
    return pl.pallas_call(
        body,
        grid=(nb,),
        in_specs=[
            pl.BlockSpec((1, 1, _BR), lambda i: (i, 0, 0)),
            pl.BlockSpec((_BR, 256), lambda i: (i, 0)),
        ],
        out_specs=[
            pl.BlockSpec((n_graphs, 256), lambda i: (0, 0)),
            pl.BlockSpec((n_graphs, 128), lambda i: (0, 0)),
        ],
        out_shape=[
            jax.ShapeDtypeStruct((n_graphs, 256), jnp.float32),
            jax.ShapeDtypeStruct((n_graphs, 128), jnp.float32),
        ],
    )(batch_r, h3)


def _head_tc(sums, cnts, gxp, w1a, w1b, b1, w2, b2, w3, b3, n_graphs):
    def body(s_ref, c_ref, gx_ref, w1a_ref, w1b_ref, b1_ref, w2_ref, b2_ref,
             w3_ref, b3_ref, o_ref):
        cnt = jnp.maximum(c_ref[:, 0:1], 1.0)
        z0 = s_ref[...] / cnt
        z1 = jnp.maximum(
            jnp.dot(z0, w1a_ref[...], preferred_element_type=jnp.float32)
            + jnp.dot(gx_ref[...], w1b_ref[...],
                      preferred_element_type=jnp.float32)
            + b1_ref[...], 0.0)
        z2 = jnp.maximum(
            jnp.dot(z1, w2_ref[...], preferred_element_type=jnp.float32)
            + b2_ref[...], 0.0)
        o_ref[...] = (jnp.dot(z2, w3_ref[...],
                              preferred_element_type=jnp.float32)
                      + b3_ref[...])

    full = lambda shape: pl.BlockSpec(shape, lambda: tuple(0 for _ in shape))
    return pl.pallas_call(
        body,
        in_specs=[
            full((n_graphs, 256)), full((n_graphs, 128)),
            full((n_graphs, 128)),
            full((256, 128)), full((128, 128)), full((1, 128)),
            full((128, 128)), full((1, 128)),
            full((128, 128)), full((1, 128)),
        ],
        out_specs=full((n_graphs, 128)),
        out_shape=jax.ShapeDtypeStruct((n_graphs, 128), jnp.float32),
    )(sums, cnts, gxp, w1a, w1b, b1, w2, b2, w3, b3)


def kernel(x, edge_index, batch, global_x, params):
    n, d = x.shape
    e = edge_index.shape[1]
    g_graphs, gd = global_x.shape

    n_pad = ((n + _BR - 1) // _BR) * _BR
    if n_pad == n:
        n_pad += _BR
    cpt = (e + 32 * _CH - 1) // (32 * _CH)
    e_pad = 32 * cpt * _CH

    f32 = jnp.float32
    xp = jnp.pad(x, ((0, n_pad - n), (0, 0)))
    srcp = jnp.concatenate(
        [edge_index[0], jnp.zeros((e_pad - e,), jnp.int32)])
    dstp = jnp.concatenate(
        [edge_index[1], jnp.full((e_pad - e,), n, jnp.int32)])
    zeros = jnp.zeros((n_pad, 128), f32)
    batch_r = jnp.pad(batch, (0, n_pad - n), constant_values=g_graphs)
    batch_r = batch_r.reshape(n_pad // _BR, 1, _BR)

    w11, b11, w12, b12 = params['mlp1']
    w21, b21, w22, b22 = params['mlp2']
    w31, b31, w32, b32 = params['mlp3']
    row = lambda v: v.reshape(1, -1)
    g1, bb1 = row(params['bn1_g']), row(params['bn1_b'])
    g2, bb2 = row(params['bn2_g']), row(params['bn2_b'])
    g3, bb3 = row(params['bn3_g']), row(params['bn3_b'])

    wf1, bf1 = params['Wf1'], row(params['bf1'])
    wf2, bf2 = params['Wf2'], row(params['bf2'])
    wf3, bf3 = params['Wf3'], row(params['bf3'])
    w1a = wf1[:256]
    w1b = jnp.pad(wf1[256:], ((0, 128 - gd), (0, 0)))
    w2p = jnp.pad(wf2, ((0, 0), (0, 128 - wf2.shape[1])))
    b2p = jnp.pad(bf2, ((0, 0), (0, 128 - bf2.shape[1])))
    w3p = jnp.pad(wf3, ((0, 128 - wf3.shape[0]), (0, 128 - wf3.shape[1])))
    b3p = jnp.pad(bf3, ((0, 0), (0, 128 - bf3.shape[1])))
    gxp = jnp.pad(global_x, ((0, 0), (0, 128 - gd)))

    p1 = _agg128(xp, srcp, dstp, zeros, n_pad, cpt)
    h1 = _layer1_tc(xp, p1, w11, row(b11), w12, row(b12), g1, bb1, n_pad)

    p2 = _agg128(h1, srcp, dstp, zeros, n_pad, cpt)
    h2 = _layer2_tc(h1, p2, w21, row(b21), w22, row(b22), g2, bb2, n_pad)

    pa = _agg128(h2[0], srcp, dstp, zeros, n_pad, cpt)
    pb = _agg128(h2[1], srcp, dstp, zeros, n_pad, cpt)
    h3 = _layer3_tc(h2, pa, pb, w31, row(b31), w32, row(b32), g3, bb3, n_pad)

    sums, cnts = _pool_tc(batch_r, h3, n_pad, g_graphs)
    out = _head_tc(sums, cnts, gxp, w1a, w1b, row(bf1), w2p, b2p, w3p, b3p,
                   g_graphs)
    return out[:, :1]


# speedup vs baseline: 1.5376x; 1.0213x over previous
import functools

import jax
import jax.numpy as jnp
from jax import lax
from jax.experimental import pallas as pl
from jax.experimental.pallas import tpu as pltpu
from jax.experimental.pallas import tpu_sc as plsc

_CH = 128
_SPLIT0 = 62
_BR = 512
_BN_C = 0.9999950000374997


def _agg128(h, srcp, dstp, zeros, n_pad, cpt):
    npt = n_pad // 16
    cpt0 = (2 * cpt * _SPLIT0 + 50) // 100
    cpt1 = 2 * cpt - cpt0
    mesh = plsc.VectorSubcoreMesh(core_axis_name="c", subcore_axis_name="s")

    @functools.partial(
        pl.kernel,
        out_type=jax.ShapeDtypeStruct((2, n_pad, 128), jnp.float32),
        mesh=mesh,
        scratch_types=[
            pltpu.VMEM((_CH,), jnp.int32),
            pltpu.VMEM((_CH,), jnp.int32),
            pltpu.VMEM((_CH, 128), jnp.float32),
            pltpu.VMEM_SHARED((n_pad, 128), jnp.float32),
            pltpu.SemaphoreType.DMA,
        ],
    )
    def k(h_hbm, src_hbm, dst_hbm, z_hbm, out_hbm,
          sidx, didx, rows, acc, gsem):
        c = lax.axis_index("c")
        s = lax.axis_index("s")
        pltpu.sync_copy(z_hbm.at[pl.ds(s * npt, npt)],
                        acc.at[pl.ds(s * npt, npt)])
        plsc.subcore_barrier()

        tb = s * (cpt0 + cpt1) + c * cpt0
        tn = jnp.where(c == 0, cpt0, cpt1)

        def body(i, carry):
            base = pl.multiple_of((tb + i) * _CH, _CH)
            pltpu.sync_copy(src_hbm.at[pl.ds(base, _CH)], sidx)
            pltpu.sync_copy(dst_hbm.at[pl.ds(base, _CH)], didx)
            pltpu.async_copy(h_hbm.at[sidx], rows, gsem).wait()
            pltpu.sync_copy(rows, acc.at[didx], add=True)
            return carry

        lax.fori_loop(0, tn, body, 0)
        plsc.subcore_barrier()
        pltpu.sync_copy(acc.at[pl.ds(s * npt, npt)],
                        out_hbm.at[c, pl.ds(s * npt, npt)])

    return k(h, srcp, dstp, zeros)


def _layer1_tc(h, p, w1, b1, w2, b2, g, bb, n_pad):
    def body(h_ref, p_ref, w1_ref, b1_ref, w2_ref, b2_ref, g_ref, bb_ref, o_ref):
        m = h_ref[...] + p_ref[0] + p_ref[1]
        t = jnp.maximum(
            jnp.dot(m, w1_ref[...], preferred_element_type=jnp.float32)
            + b1_ref[...], 0.0)
        u = (jnp.dot(t, w2_ref[...], preferred_element_type=jnp.float32)
             + b2_ref[...])
        o_ref[...] = (jnp.maximum(u, 0.0) * (g_ref[...] * _BN_C) + bb_ref[...])

    nb = n_pad // _BR
    return pl.pallas_call(
        body,
        grid=(nb,),
        in_specs=[
            pl.BlockSpec((_BR, 128), lambda i: (i, 0)),
            pl.BlockSpec((2, _BR, 128), lambda i: (0, i, 0)),
            pl.BlockSpec((128, 128), lambda i: (0, 0)),
            pl.BlockSpec((1, 128), lambda i: (0, 0)),
            pl.BlockSpec((128, 128), lambda i: (0, 0)),
            pl.BlockSpec((1, 128), lambda i: (0, 0)),
            pl.BlockSpec((1, 128), lambda i: (0, 0)),
            pl.BlockSpec((1, 128), lambda i: (0, 0)),
        ],
        out_specs=pl.BlockSpec((_BR, 128), lambda i: (i, 0)),
        out_shape=jax.ShapeDtypeStruct((n_pad, 128), jnp.float32),
    )(h, p, w1, b1, w2, b2, g, bb)


def _layer2_tc(h, p, w1, b1, w2, b2, g, bb, n_pad):
    def body(h_ref, p_ref, w1_ref, b1_ref, w2_ref, b2_ref, g_ref, bb_ref, o_ref):
        m = h_ref[...] + p_ref[0] + p_ref[1]
        t = jnp.maximum(
            jnp.dot(m, w1_ref[...], preferred_element_type=jnp.float32)
            + b1_ref[...], 0.0)
        u = (jnp.dot(t, w2_ref[...], preferred_element_type=jnp.float32)
             + b2_ref[...])
        h2 = jnp.maximum(u, 0.0) * (g_ref[...] * _BN_C) + bb_ref[...]
        o_ref[0] = h2[:, :128]
        o_ref[1] = h2[:, 128:]

    nb = n_pad // _BR
    return pl.pallas_call(
        body,
        grid=(nb,),
        in_specs=[
            pl.BlockSpec((_BR, 128), lambda i: (i, 0)),
            pl.BlockSpec((2, _BR, 128), lambda i: (0, i, 0)),
            pl.BlockSpec((128, 256), lambda i: (0, 0)),
            pl.BlockSpec((1, 256), lambda i: (0, 0)),
            pl.BlockSpec((256, 256), lambda i: (0, 0)),
            pl.BlockSpec((1, 256), lambda i: (0, 0)),
            pl.BlockSpec((1, 256), lambda i: (0, 0)),
            pl.BlockSpec((1, 256), lambda i: (0, 0)),
        ],
        out_specs=pl.BlockSpec((2, _BR, 128), lambda i: (0, i, 0)),
        out_shape=jax.ShapeDtypeStruct((2, n_pad, 128), jnp.float32),
    )(h, p, w1, b1, w2, b2, g, bb)


def _layer3_tc(hh, pa, pb, w1, b1, w2, b2, g, bb, n_pad):
    def body(hh_ref, pa_ref, pb_ref, w1_ref, b1_ref, w2_ref, b2_ref,
             g_ref, bb_ref, o_ref):
        m0 = hh_ref[0] + pa_ref[0] + pa_ref[1]
        m1 = hh_ref[1] + pb_ref[0] + pb_ref[1]
        m = jnp.concatenate([m0, m1], axis=1)
        t = jnp.maximum(
            jnp.dot(m, w1_ref[...], preferred_element_type=jnp.float32)
            + b1_ref[...], 0.0)
        u = (jnp.dot(t, w2_ref[...], preferred_element_type=jnp.float32)
             + b2_ref[...])
        o_ref[...] = jnp.maximum(u, 0.0) * (g_ref[...] * _BN_C) + bb_ref[...]

    nb = n_pad // _BR
    return pl.pallas_call(
        body,
        grid=(nb,),
        in_specs=[
            pl.BlockSpec((2, _BR, 128), lambda i: (0, i, 0)),
            pl.BlockSpec((2, _BR, 128), lambda i: (0, i, 0)),
            pl.BlockSpec((2, _BR, 128), lambda i: (0, i, 0)),
            pl.BlockSpec((256, 256), lambda i: (0, 0)),
            pl.BlockSpec((1, 256), lambda i: (0, 0)),
            pl.BlockSpec((256, 256), lambda i: (0, 0)),
            pl.BlockSpec((1, 256), lambda i: (0, 0)),
            pl.BlockSpec((1, 256), lambda i: (0, 0)),
            pl.BlockSpec((1, 256), lambda i: (0, 0)),
        ],
        out_specs=pl.BlockSpec((_BR, 256), lambda i: (i, 0)),
        out_shape=jax.ShapeDtypeStruct((n_pad, 256), jnp.float32),
    )(hh, pa, pb, w1, b1, w2, b2, g, bb)


def _pool_tc(batch_r, h3, n_pad, n_graphs):
    def body(b_ref, h_ref, s_ref, c_ref):
        i = pl.program_id(0)

        @pl.when(i == 0)
        def _():
            s_ref[...] = jnp.zeros_like(s_ref)
            c_ref[...] = jnp.zeros_like(c_ref)

        bvals = b_ref[0]
        ids = lax.broadcasted_iota(jnp.int32, (n_graphs, _BR), 0)
        oh = (ids == bvals).astype(jnp.float32)
        s_ref[...] += jnp.dot(oh, h_ref[...],
                              preferred_element_type=jnp.float32)
        cnt = jnp.sum(oh, axis=1, keepdims=True)
        c_ref[...] += jnp.broadcast_to(cnt, (n_graphs, 128))

    nb = n_pad // _BR
    return pl.pallas_call(
        body,
        grid=(nb,),
        in_specs=[
            pl.BlockSpec((1, 1, _BR), lambda i: (i, 0, 0)),
            pl.BlockSpec((_BR, 256), lambda i: (i, 0)),
        ],
        out_specs=[
            pl.BlockSpec((n_graphs, 256), lambda i: (0, 0)),
            pl.BlockSpec((n_graphs, 128), lambda i: (0, 0)),
        ],
        out_shape=[
            jax.ShapeDtypeStruct((n_graphs, 256), jnp.float32),
            jax.ShapeDtypeStruct((n_graphs, 128), jnp.float32),
        ],
    )(batch_r, h3)


def _head_tc(sums, cnts, gxp, w1a, w1b, b1, w2, b2, w3, b3, n_graphs):
    def body(s_ref, c_ref, gx_ref, w1a_ref, w1b_ref, b1_ref, w2_ref, b2_ref,
             w3_ref, b3_ref, o_ref):
        cnt = jnp.maximum(c_ref[:, 0:1], 1.0)
        z0 = s_ref[...] / cnt
        z1 = jnp.maximum(
            jnp.dot(z0, w1a_ref[...], preferred_element_type=jnp.float32)
            + jnp.dot(gx_ref[...], w1b_ref[...],
                      preferred_element_type=jnp.float32)
            + b1_ref[...], 0.0)
        z2 = jnp.maximum(
            jnp.dot(z1, w2_ref[...], preferred_element_type=jnp.float32)
            + b2_ref[...], 0.0)
        o_ref[...] = (jnp.dot(z2, w3_ref[...],
                              preferred_element_type=jnp.float32)
                      + b3_ref[...])

    full = lambda shape: pl.BlockSpec(shape, lambda: tuple(0 for _ in shape))
    return pl.pallas_call(
        body,
        in_specs=[
            full((n_graphs, 256)), full((n_graphs, 128)),
            full((n_graphs, 128)),
            full((256, 128)), full((128, 128)), full((1, 128)),
            full((128, 128)), full((1, 128)),
            full((128, 128)), full((1, 128)),
        ],
        out_specs=full((n_graphs, 128)),
        out_shape=jax.ShapeDtypeStruct((n_graphs, 128), jnp.float32),
    )(sums, cnts, gxp, w1a, w1b, b1, w2, b2, w3, b3)


def kernel(x, edge_index, batch, global_x, params):
    n, d = x.shape
    e = edge_index.shape[1]
    g_graphs, gd = global_x.shape

    n_pad = ((n + _BR - 1) // _BR) * _BR
    if n_pad == n:
        n_pad += _BR
    cpt = (e + 32 * _CH - 1) // (32 * _CH)
    e_pad = 32 * cpt * _CH

    f32 = jnp.float32
    xp = jnp.pad(x, ((0, n_pad - n), (0, 0)))
    srcp = jnp.concatenate(
        [edge_index[0], jnp.zeros((e_pad - e,), jnp.int32)])
    dstp = jnp.concatenate(
        [edge_index[1], jnp.full((e_pad - e,), n, jnp.int32)])
    zeros = jnp.zeros((n_pad, 128), f32)
    batch_r = jnp.pad(batch, (0, n_pad - n), constant_values=g_graphs)
    batch_r = batch_r.reshape(n_pad // _BR, 1, _BR)

    w11, b11, w12, b12 = params['mlp1']
    w21, b21, w22, b22 = params['mlp2']
    w31, b31, w32, b32 = params['mlp3']
    row = lambda v: v.reshape(1, -1)
    g1, bb1 = row(params['bn1_g']), row(params['bn1_b'])
    g2, bb2 = row(params['bn2_g']), row(params['bn2_b'])
    g3, bb3 = row(params['bn3_g']), row(params['bn3_b'])

    wf1, bf1 = params['Wf1'], row(params['bf1'])
    wf2, bf2 = params['Wf2'], row(params['bf2'])
    wf3, bf3 = params['Wf3'], row(params['bf3'])
    w1a = wf1[:256]
    w1b = jnp.pad(wf1[256:], ((0, 128 - gd), (0, 0)))
    w2p = jnp.pad(wf2, ((0, 0), (0, 128 - wf2.shape[1])))
    b2p = jnp.pad(bf2, ((0, 0), (0, 128 - bf2.shape[1])))
    w3p = jnp.pad(wf3, ((0, 128 - wf3.shape[0]), (0, 128 - wf3.shape[1])))
    b3p = jnp.pad(bf3, ((0, 0), (0, 128 - bf3.shape[1])))
    gxp = jnp.pad(global_x, ((0, 0), (0, 128 - gd)))

    p1 = _agg128(xp, srcp, dstp, zeros, n_pad, cpt)
    h1 = _layer1_tc(xp, p1, w11, row(b11), w12, row(b12), g1, bb1, n_pad)

    p2 = _agg128(h1, srcp, dstp, zeros, n_pad, cpt)
    h2 = _layer2_tc(h1, p2, w21, row(b21), w22, row(b22), g2, bb2, n_pad)

    pa = _agg128(h2[0], srcp, dstp, zeros, n_pad, cpt)
    pb = _agg128(h2[1], srcp, dstp, zeros, n_pad, cpt)
    h3 = _layer3_tc(h2, pa, pb, w31, row(b31), w32, row(b32), g3, bb3, n_pad)

    sums, cnts = _pool_tc(batch_r, h3, n_pad, g_graphs)
    out = _head_tc(sums, cnts, gxp, w1a, w1b, row(bf1), w2p, b2p, w3p, b3p,
                   g_graphs)
    return out[:, :1]


# speedup vs baseline: 1.5468x; 1.0060x over previous
import functools

import jax
import jax.numpy as jnp
from jax import lax
from jax.experimental import pallas as pl
from jax.experimental.pallas import tpu as pltpu
from jax.experimental.pallas import tpu_sc as plsc

_CH = 128
_SPLIT0 = 65
_BR = 512
_BN_C = 0.9999950000374997


def _agg128(h, srcp, dstp, zeros, n_pad, cpt):
    npt = n_pad // 16
    cpt0 = (2 * cpt * _SPLIT0 + 50) // 100
    cpt1 = 2 * cpt - cpt0
    mesh = plsc.VectorSubcoreMesh(core_axis_name="c", subcore_axis_name="s")

    @functools.partial(
        pl.kernel,
        out_type=jax.ShapeDtypeStruct((2, n_pad, 128), jnp.float32),
        mesh=mesh,
        scratch_types=[
            pltpu.VMEM((_CH,), jnp.int32),
            pltpu.VMEM((_CH,), jnp.int32),
            pltpu.VMEM((_CH, 128), jnp.float32),
            pltpu.VMEM_SHARED((n_pad, 128), jnp.float32),
            pltpu.SemaphoreType.DMA,
        ],
    )
    def k(h_hbm, src_hbm, dst_hbm, z_hbm, out_hbm,
          sidx, didx, rows, acc, gsem):
        c = lax.axis_index("c")
        s = lax.axis_index("s")
        pltpu.sync_copy(z_hbm.at[pl.ds(s * npt, npt)],
                        acc.at[pl.ds(s * npt, npt)])
        plsc.subcore_barrier()

        tb = s * (cpt0 + cpt1) + c * cpt0
        tn = jnp.where(c == 0, cpt0, cpt1)

        def body(i, carry):
            base = pl.multiple_of((tb + i) * _CH, _CH)
            pltpu.sync_copy(src_hbm.at[pl.ds(base, _CH)], sidx)
            pltpu.sync_copy(dst_hbm.at[pl.ds(base, _CH)], didx)
            pltpu.async_copy(h_hbm.at[sidx], rows, gsem).wait()
            pltpu.sync_copy(rows, acc.at[didx], add=True)
            return carry

        lax.fori_loop(0, tn, body, 0)
        plsc.subcore_barrier()
        pltpu.sync_copy(acc.at[pl.ds(s * npt, npt)],
                        out_hbm.at[c, pl.ds(s * npt, npt)])

    return k(h, srcp, dstp, zeros)


def _layer1_tc(h, p, w1, b1, w2, b2, g, bb, n_pad):
    def body(h_ref, p_ref, w1_ref, b1_ref, w2_ref, b2_ref, g_ref, bb_ref, o_ref):
        m = h_ref[...] + p_ref[0] + p_ref[1]
        t = jnp.maximum(
            jnp.dot(m, w1_ref[...], preferred_element_type=jnp.float32)
            + b1_ref[...], 0.0)
        u = (jnp.dot(t, w2_ref[...], preferred_element_type=jnp.float32)
             + b2_ref[...])
        o_ref[...] = (jnp.maximum(u, 0.0) * (g_ref[...] * _BN_C) + bb_ref[...])

    nb = n_pad // _BR
    return pl.pallas_call(
        body,
        grid=(nb,),
        in_specs=[
            pl.BlockSpec((_BR, 128), lambda i: (i, 0)),
            pl.BlockSpec((2, _BR, 128), lambda i: (0, i, 0)),
            pl.BlockSpec((128, 128), lambda i: (0, 0)),
            pl.BlockSpec((1, 128), lambda i: (0, 0)),
            pl.BlockSpec((128, 128), lambda i: (0, 0)),
            pl.BlockSpec((1, 128), lambda i: (0, 0)),
            pl.BlockSpec((1, 128), lambda i: (0, 0)),
            pl.BlockSpec((1, 128), lambda i: (0, 0)),
        ],
        out_specs=pl.BlockSpec((_BR, 128), lambda i: (i, 0)),
        out_shape=jax.ShapeDtypeStruct((n_pad, 128), jnp.float32),
    )(h, p, w1, b1, w2, b2, g, bb)


def _layer2_tc(h, p, w1, b1, w2, b2, g, bb, n_pad):
    def body(h_ref, p_ref, w1_ref, b1_ref, w2_ref, b2_ref, g_ref, bb_ref, o_ref):
        m = h_ref[...] + p_ref[0] + p_ref[1]
        t = jnp.maximum(
            jnp.dot(m, w1_ref[...], preferred_element_type=jnp.float32)
            + b1_ref[...], 0.0)
        u = (jnp.dot(t, w2_ref[...], preferred_element_type=jnp.float32)
             + b2_ref[...])
        h2 = jnp.maximum(u, 0.0) * (g_ref[...] * _BN_C) + bb_ref[...]
        o_ref[0] = h2[:, :128]
        o_ref[1] = h2[:, 128:]

    nb = n_pad // _BR
    return pl.pallas_call(
        body,
        grid=(nb,),
        in_specs=[
            pl.BlockSpec((_BR, 128), lambda i: (i, 0)),
            pl.BlockSpec((2, _BR, 128), lambda i: (0, i, 0)),
            pl.BlockSpec((128, 256), lambda i: (0, 0)),
            pl.BlockSpec((1, 256), lambda i: (0, 0)),
            pl.BlockSpec((256, 256), lambda i: (0, 0)),
            pl.BlockSpec((1, 256), lambda i: (0, 0)),
            pl.BlockSpec((1, 256), lambda i: (0, 0)),
            pl.BlockSpec((1, 256), lambda i: (0, 0)),
        ],
        out_specs=pl.BlockSpec((2, _BR, 128), lambda i: (0, i, 0)),
        out_shape=jax.ShapeDtypeStruct((2, n_pad, 128), jnp.float32),
    )(h, p, w1, b1, w2, b2, g, bb)


def _layer3_tc(hh, pa, pb, w1, b1, w2, b2, g, bb, n_pad):
    def body(hh_ref, pa_ref, pb_ref, w1_ref, b1_ref, w2_ref, b2_ref,
             g_ref, bb_ref, o_ref):
        m0 = hh_ref[0] + pa_ref[0] + pa_ref[1]
        m1 = hh_ref[1] + pb_ref[0] + pb_ref[1]
        m = jnp.concatenate([m0, m1], axis=1)
        t = jnp.maximum(
            jnp.dot(m, w1_ref[...], preferred_element_type=jnp.float32)
            + b1_ref[...], 0.0)
        u = (jnp.dot(t, w2_ref[...], preferred_element_type=jnp.float32)
             + b2_ref[...])
        o_ref[...] = jnp.maximum(u, 0.0) * (g_ref[...] * _BN_C) + bb_ref[...]

    nb = n_pad // _BR
    return pl.pallas_call(
        body,
        grid=(nb,),
        in_specs=[
            pl.BlockSpec((2, _BR, 128), lambda i: (0, i, 0)),
            pl.BlockSpec((2, _BR, 128), lambda i: (0, i, 0)),
            pl.BlockSpec((2, _BR, 128), lambda i: (0, i, 0)),
            pl.BlockSpec((256, 256), lambda i: (0, 0)),
            pl.BlockSpec((1, 256), lambda i: (0, 0)),
            pl.BlockSpec((256, 256), lambda i: (0, 0)),
            pl.BlockSpec((1, 256), lambda i: (0, 0)),
            pl.BlockSpec((1, 256), lambda i: (0, 0)),
            pl.BlockSpec((1, 256), lambda i: (0, 0)),
        ],
        out_specs=pl.BlockSpec((_BR, 256), lambda i: (i, 0)),
        out_shape=jax.ShapeDtypeStruct((n_pad, 256), jnp.float32),
    )(hh, pa, pb, w1, b1, w2, b2, g, bb)


def _pool_tc(batch_r, h3, n_pad, n_graphs):
    def body(b_ref, h_ref, s_ref, c_ref):
        i = pl.program_id(0)

        @pl.when(i == 0)
        def _():
            s_ref[...] = jnp.zeros_like(s_ref)
            c_ref[...] = jnp.zeros_like(c_ref)

        bvals = b_ref[0]
        ids = lax.broadcasted_iota(jnp.int32, (n_graphs, _BR), 0)
        oh = (ids == bvals).astype(jnp.float32)
        s_ref[...] += jnp.dot(oh, h_ref[...],
                              preferred_element_type=jnp.float32)
        cnt = jnp.sum(oh, axis=1, keepdims=True)
        c_ref[...] += jnp.broadcast_to(cnt, (n_graphs, 128))

    nb = n_pad // _BR
    return pl.pallas_call(
        body,
        grid=(nb,),
        in_specs=[
            pl.BlockSpec((1, 1, _BR), lambda i: (i, 0, 0)),
            pl.BlockSpec((_BR, 256), lambda i: (i, 0)),
        ],
        out_specs=[
            pl.BlockSpec((n_graphs, 256), lambda i: (0, 0)),
            pl.BlockSpec((n_graphs, 128), lambda i: (0, 0)),
        ],
        out_shape=[
            jax.ShapeDtypeStruct((n_graphs, 256), jnp.float32),
            jax.ShapeDtypeStruct((n_graphs, 128), jnp.float32),
        ],
    )(batch_r, h3)


def _head_tc(sums, cnts, gxp, w1a, w1b, b1, w2, b2, w3, b3, n_graphs):
    def body(s_ref, c_ref, gx_ref, w1a_ref, w1b_ref, b1_ref, w2_ref, b2_ref,
             w3_ref, b3_ref, o_ref):
        cnt = jnp.maximum(c_ref[:, 0:1], 1.0)
        z0 = s_ref[...] / cnt
        z1 = jnp.maximum(
            jnp.dot(z0, w1a_ref[...], preferred_element_type=jnp.float32)
            + jnp.dot(gx_ref[...], w1b_ref[...],
                      preferred_element_type=jnp.float32)
            + b1_ref[...], 0.0)
        z2 = jnp.maximum(
            jnp.dot(z1, w2_ref[...], preferred_element_type=jnp.float32)
            + b2_ref[...], 0.0)
        o_ref[...] = (jnp.dot(z2, w3_ref[...],
                              preferred_element_type=jnp.float32)
                      + b3_ref[...])

    full = lambda shape: pl.BlockSpec(shape, lambda: tuple(0 for _ in shape))
    return pl.pallas_call(
        body,
        in_specs=[
            full((n_graphs, 256)), full((n_graphs, 128)),
            full((n_graphs, 128)),
            full((256, 128)), full((128, 128)), full((1, 128)),
            full((128, 128)), full((1, 128)),
            full((128, 128)), full((1, 128)),
        ],
        out_specs=full((n_graphs, 128)),
        out_shape=jax.ShapeDtypeStruct((n_graphs, 128), jnp.float32),
    )(sums, cnts, gxp, w1a, w1b, b1, w2, b2, w3, b3)


def kernel(x, edge_index, batch, global_x, params):
    n, d = x.shape
    e = edge_index.shape[1]
    g_graphs, gd = global_x.shape

    n_pad = ((n + _BR - 1) // _BR) * _BR
    if n_pad == n:
        n_pad += _BR
    cpt = (e + 32 * _CH - 1) // (32 * _CH)
    e_pad = 32 * cpt * _CH

    f32 = jnp.float32
    xp = jnp.pad(x, ((0, n_pad - n), (0, 0)))
    srcp = jnp.concatenate(
        [edge_index[0], jnp.zeros((e_pad - e,), jnp.int32)])
    dstp = jnp.concatenate(
        [edge_index[1], jnp.full((e_pad - e,), n, jnp.int32)])
    zeros = jnp.zeros((n_pad, 128), f32)
    batch_r = jnp.pad(batch, (0, n_pad - n), constant_values=g_graphs)
    batch_r = batch_r.reshape(n_pad // _BR, 1, _BR)

    w11, b11, w12, b12 = params['mlp1']
    w21, b21, w22, b22 = params['mlp2']
    w31, b31, w32, b32 = params['mlp3']
    row = lambda v: v.reshape(1, -1)
    g1, bb1 = row(params['bn1_g']), row(params['bn1_b'])
    g2, bb2 = row(params['bn2_g']), row(params['bn2_b'])
    g3, bb3 = row(params['bn3_g']), row(params['bn3_b'])

    wf1, bf1 = params['Wf1'], row(params['bf1'])
    wf2, bf2 = params['Wf2'], row(params['bf2'])
    wf3, bf3 = params['Wf3'], row(params['bf3'])
    w1a = wf1[:256]
    w1b = jnp.pad(wf1[256:], ((0, 128 - gd), (0, 0)))
    w2p = jnp.pad(wf2, ((0, 0), (0, 128 - wf2.shape[1])))
    b2p = jnp.pad(bf2, ((0, 0), (0, 128 - bf2.shape[1])))
    w3p = jnp.pad(wf3, ((0, 128 - wf3.shape[0]), (0, 128 - wf3.shape[1])))
    b3p = jnp.pad(bf3, ((0, 0), (0, 128 - bf3.shape[1])))
    gxp = jnp.pad(global_x, ((0, 0), (0, 128 - gd)))

    p1 = _agg128(xp, srcp, dstp, zeros, n_pad, cpt)
    h1 = _layer1_tc(xp, p1, w11, row(b11), w12, row(b12), g1, bb1, n_pad)

    p2 = _agg128(h1, srcp, dstp, zeros, n_pad, cpt)
    h2 = _layer2_tc(h1, p2, w21, row(b21), w22, row(b22), g2, bb2, n_pad)

    pa = _agg128(h2[0], srcp, dstp, zeros, n_pad, cpt)
    pb = _agg128(h2[1], srcp, dstp, zeros, n_pad, cpt)
    h3 = _layer3_tc(h2, pa, pb, w31, row(b31), w32, row(b32), g3, bb3, n_pad)

    sums, cnts = _pool_tc(batch_r, h3, n_pad, g_graphs)
    out = _head_tc(sums, cnts, gxp, w1a, w1b, row(bf1), w2p, b2p, w3p, b3p,
                   g_graphs)
    return out[:, :1]
